# Initial kernel scaffold; baseline (speedup 1.0000x reference)
import jax
import jax.numpy as jnp
from jax.experimental import pallas as pl


def kernel(features, edge_index, W1, b1, W2, b2):
    raise NotImplementedError("write your pallas kernel here")


# speedup vs baseline: 4.5834x; 4.5834x over previous
import functools

import jax
import jax.numpy as jnp
from jax import lax
from jax.experimental import pallas as pl
from jax.experimental.pallas import tpu as pltpu
from jax.experimental.pallas import tpu_sc as plsc

N = 10000
E = 320000
D1 = 128
D2 = 16

NC, NS = 2, 16
NW = NC * NS
CH = 80
EPW = E // NW
NCHUNK = EPW // CH
RPT = N // NS
ZR = 125
NZ = RPT // ZR
NBLK = N // ZR

_MESH = plsc.VectorSubcoreMesh(core_axis_name="c", subcore_axis_name="s")
_SC_PARAMS = pltpu.CompilerParams(use_tc_tiling_on_sc=False)


def _zero_buf(buf, rows, d):
    z16 = jnp.zeros((16,), jnp.float32)

    def zrow(r, carry):
        for c8 in range(d // 16):
            buf[r, pl.ds(c8 * 16, 16)] = z16
        return carry

    lax.fori_loop(0, rows, zrow, 0)


@functools.partial(
    pl.kernel,
    out_type=jax.ShapeDtypeStruct((NC, 2, NBLK, ZR, 16), jnp.float32),
    mesh=_MESH,
    compiler_params=_SC_PARAMS,
    scratch_types=[
        pltpu.VMEM_SHARED((N, 16), jnp.float32),
        pltpu.VMEM_SHARED((N, 16), jnp.float32),
        pltpu.VMEM((CH, 16), jnp.float32),
        pltpu.VMEM((ZR, 16), jnp.float32),
        pltpu.VMEM((CH,), jnp.int32),
        pltpu.VMEM((CH,), jnp.int32),
    ],
)
def _sc_degrees(src_hbm, dst_hbm, out_hbm, do_sh, di_sh, ones_v, zb_v, isrc_v, idst_v):
    cid = lax.axis_index("c")
    sid = lax.axis_index("s")
    wid = sid * NC + cid

    one16 = jnp.ones((16,), jnp.float32)

    def orow(r, carry):
        ones_v[r, pl.ds(0, 16)] = one16
        return carry

    lax.fori_loop(0, CH, orow, 0)
    _zero_buf(zb_v, ZR, 16)

    def zcp(k, carry):
        r0 = sid * RPT + k * ZR
        pltpu.sync_copy(zb_v, do_sh.at[pl.ds(r0, ZR)])
        pltpu.sync_copy(zb_v, di_sh.at[pl.ds(r0, ZR)])
        return carry

    lax.fori_loop(0, NZ, zcp, 0)
    plsc.subcore_barrier()

    ebase = wid * EPW

    def body(ci, carry):
        b = ebase + ci * CH
        pltpu.sync_copy(src_hbm.at[pl.ds(b, CH)], isrc_v)
        pltpu.sync_copy(dst_hbm.at[pl.ds(b, CH)], idst_v)
        pltpu.sync_copy(ones_v, do_sh.at[isrc_v], add=True)
        pltpu.sync_copy(ones_v, di_sh.at[idst_v], add=True)
        return carry

    lax.fori_loop(0, NCHUNK, body, 0)
    plsc.subcore_barrier()

    def cpo(k, carry):
        r0 = sid * RPT + k * ZR
        blk = sid * NZ + k
        pltpu.sync_copy(do_sh.at[pl.ds(r0, ZR)], zb_v)
        pltpu.sync_copy(zb_v, out_hbm.at[cid, 0, blk])
        pltpu.sync_copy(di_sh.at[pl.ds(r0, ZR)], zb_v)
        pltpu.sync_copy(zb_v, out_hbm.at[cid, 1, blk])
        return carry

    lax.fori_loop(0, NZ, cpo, 0)


def _make_sc_pass(d):
    @functools.partial(
        pl.kernel,
        out_type=jax.ShapeDtypeStruct((NC, NBLK, ZR, d), jnp.float32),
        mesh=_MESH,
        compiler_params=_SC_PARAMS,
        scratch_types=[
            pltpu.VMEM_SHARED((N, d), jnp.float32),
            pltpu.VMEM((CH, d), jnp.float32),
            pltpu.VMEM((ZR, d), jnp.float32),
            pltpu.VMEM((CH,), jnp.int32),
            pltpu.VMEM((CH,), jnp.int32),
            pltpu.SemaphoreType.DMA,
        ],
    )
    def k(y_hbm, src_hbm, dst_hbm, out_hbm, acc_sh, rows_v, zb_v, isrc_v, idst_v, sem):
        cid = lax.axis_index("c")
        sid = lax.axis_index("s")
        wid = sid * NC + cid

        _zero_buf(zb_v, ZR, d)

        def zcp(kk, carry):
            pltpu.sync_copy(zb_v, acc_sh.at[pl.ds(sid * RPT + kk * ZR, ZR)])
            return carry

        lax.fori_loop(0, NZ, zcp, 0)
        plsc.subcore_barrier()

        ebase = wid * EPW

        def body(ci, carry):
            b = ebase + ci * CH
            pltpu.sync_copy(src_hbm.at[pl.ds(b, CH)], isrc_v)
            pltpu.sync_copy(dst_hbm.at[pl.ds(b, CH)], idst_v)
            pltpu.async_copy(y_hbm.at[isrc_v], rows_v, sem).wait()
            pltpu.sync_copy(rows_v, acc_sh.at[idst_v], add=True)
            return carry

        lax.fori_loop(0, NCHUNK, body, 0)
        plsc.subcore_barrier()

        def cpo(kk, carry):
            r0 = sid * RPT + kk * ZR
            blk = sid * NZ + kk
            pltpu.sync_copy(acc_sh.at[pl.ds(r0, ZR)], zb_v)
            pltpu.sync_copy(zb_v, out_hbm.at[cid, blk])
            return carry

        lax.fori_loop(0, NZ, cpo, 0)

    return k


_sc_pass128 = _make_sc_pass(D1)
_sc_pass16 = _make_sc_pass(D2)

_BM = 1000


def _ns_from(d_ref, which):
    cnt = d_ref[0, which][:, :1] + d_ref[1, which][:, :1]
    return lax.rsqrt(jnp.maximum(cnt, 1.0))


def _tc1(x, W1, degp):
    def body(x_ref, w_ref, d_ref, o_ref):
        ns = _ns_from(d_ref, 0)
        o_ref[...] = jnp.dot(x_ref[...] * ns, w_ref[...],
                             preferred_element_type=jnp.float32)

    return pl.pallas_call(
        body,
        grid=(N // _BM,),
        in_specs=[
            pl.BlockSpec((_BM, D1), lambda m: (m, 0)),
            pl.BlockSpec((D1, D1), lambda m: (0, 0)),
            pl.BlockSpec((2, 2, _BM, 16), lambda m: (0, 0, m, 0)),
        ],
        out_specs=pl.BlockSpec((_BM, D1), lambda m: (m, 0)),
        out_shape=jax.ShapeDtypeStruct((N, D1), jnp.float32),
    )(x, W1, degp)


def _tc2(aggp, degp, b1r, W2):
    def body(a_ref, d_ref, b_ref, w_ref, o_ref):
        agg = a_ref[0] + a_ref[1]
        nd = _ns_from(d_ref, 1)
        ns = _ns_from(d_ref, 0)
        h = jnp.maximum(agg * nd + b_ref[...], 0.0)
        o_ref[...] = jnp.dot(h * ns, w_ref[...],
                             preferred_element_type=jnp.float32)

    return pl.pallas_call(
        body,
        grid=(N // _BM,),
        in_specs=[
            pl.BlockSpec((2, _BM, D1), lambda m: (0, m, 0)),
            pl.BlockSpec((2, 2, _BM, 16), lambda m: (0, 0, m, 0)),
            pl.BlockSpec((1, D1), lambda m: (0, 0)),
            pl.BlockSpec((D1, D2), lambda m: (0, 0)),
        ],
        out_specs=pl.BlockSpec((_BM, D2), lambda m: (m, 0)),
        out_shape=jax.ShapeDtypeStruct((N, D2), jnp.float32),
    )(aggp, degp, b1r, W2)


def _tc3(aggp, degp, b2r):
    def body(a_ref, d_ref, b_ref, o_ref):
        agg = a_ref[0] + a_ref[1]
        nd = _ns_from(d_ref, 1)
        o_ref[...] = agg * nd + b_ref[...]

    return pl.pallas_call(
        body,
        grid=(N // _BM,),
        in_specs=[
            pl.BlockSpec((2, _BM, D2), lambda m: (0, m, 0)),
            pl.BlockSpec((2, 2, _BM, 16), lambda m: (0, 0, m, 0)),
            pl.BlockSpec((1, D2), lambda m: (0, 0)),
        ],
        out_specs=pl.BlockSpec((_BM, D2), lambda m: (m, 0)),
        out_shape=jax.ShapeDtypeStruct((N, D2), jnp.float32),
    )(aggp, degp, b2r)


def kernel(features, edge_index, W1, b1, W2, b2):
    src = edge_index[0]
    dst = edge_index[1]
    deg_parts = _sc_degrees(src, dst).reshape(NC, 2, N, 16)
    y1s = _tc1(features, W1, deg_parts)
    agg1 = _sc_pass128(y1s, src, dst).reshape(NC, N, D1)
    y2s = _tc2(agg1, deg_parts, b1.reshape(1, D1), W2)
    agg2 = _sc_pass16(y2s, src, dst).reshape(NC, N, D2)
    return _tc3(agg2, deg_parts, b2.reshape(1, D2))


# speedup vs baseline: 11.7288x; 2.5590x over previous
import functools

import jax
import jax.numpy as jnp
from jax import lax
from jax.experimental import pallas as pl
from jax.experimental.pallas import tpu as pltpu
from jax.experimental.pallas import tpu_sc as plsc

N = 10000
E = 320000
D1 = 128
D2 = 16

NC, NS = 2, 16
NW = NC * NS
CH = 40
EPW = E // NW
NCHUNK = EPW // CH
RPT = N // NS
ZR = 25
NZ = RPT // ZR
K = 5

_MESH = plsc.VectorSubcoreMesh(core_axis_name="c", subcore_axis_name="s")
_SC_PARAMS = pltpu.CompilerParams(use_tc_tiling_on_sc=False)


def _zero_buf(buf, rows, d):
    z16 = jnp.zeros((16,), jnp.float32)

    def zrow(r, carry):
        for c8 in range(d // 16):
            buf[r, pl.ds(c8 * 16, 16)] = z16
        return carry

    lax.fori_loop(0, rows, zrow, 0)


@functools.partial(
    pl.kernel,
    out_type=jax.ShapeDtypeStruct((NC, 2, NS, RPT, 16), jnp.float32),
    mesh=_MESH,
    compiler_params=_SC_PARAMS,
    scratch_types=[
        pltpu.VMEM_SHARED((N, 16), jnp.float32),
        pltpu.VMEM_SHARED((N, 16), jnp.float32),
        pltpu.VMEM((CH, 16), jnp.float32),
        pltpu.VMEM((ZR, 16), jnp.float32),
        pltpu.VMEM((NCHUNK, CH), jnp.int32),
        pltpu.VMEM((NCHUNK, CH), jnp.int32),
        pltpu.SemaphoreType.DMA,
    ],
)
def _sc_degrees(src_hbm, dst_hbm, out_hbm, do_sh, di_sh, ones_v, zb_v, isrc_v, idst_v, sem):
    cid = lax.axis_index("c")
    sid = lax.axis_index("s")
    wid = sid * NC + cid

    one16 = jnp.ones((16,), jnp.float32)

    def orow(r, carry):
        ones_v[r, pl.ds(0, 16)] = one16
        return carry

    lax.fori_loop(0, CH, orow, 0)
    _zero_buf(zb_v, ZR, 16)

    pltpu.sync_copy(src_hbm.at[wid], isrc_v)
    pltpu.sync_copy(dst_hbm.at[wid], idst_v)

    def zcp(k, carry):
        r0 = sid * RPT + k * ZR
        pltpu.sync_copy(zb_v, do_sh.at[pl.ds(r0, ZR)])
        pltpu.sync_copy(zb_v, di_sh.at[pl.ds(r0, ZR)])
        return carry

    lax.fori_loop(0, NZ, zcp, 0)
    plsc.subcore_barrier()

    def fire(ci, carry):
        pltpu.async_copy(ones_v, do_sh.at[isrc_v.at[ci]], sem, add=True)
        pltpu.async_copy(ones_v, di_sh.at[idst_v.at[ci]], sem, add=True)
        return carry

    lax.fori_loop(0, NCHUNK, fire, 0)

    def drain(ci, carry):
        pltpu.make_async_copy(ones_v, do_sh.at[isrc_v.at[0]], sem).wait()
        return carry

    lax.fori_loop(0, 2 * NCHUNK, drain, 0)
    plsc.subcore_barrier()

    pltpu.sync_copy(do_sh.at[pl.ds(sid * RPT, RPT)], out_hbm.at[cid, 0, sid])
    pltpu.sync_copy(di_sh.at[pl.ds(sid * RPT, RPT)], out_hbm.at[cid, 1, sid])


def _make_sc_pass(d):
    @functools.partial(
        pl.kernel,
        out_type=jax.ShapeDtypeStruct((NC, NS, RPT, d), jnp.float32),
        mesh=_MESH,
        compiler_params=_SC_PARAMS,
        scratch_types=[
            pltpu.VMEM_SHARED((N, d), jnp.float32),
            pltpu.VMEM((K, CH, d), jnp.float32),
            pltpu.VMEM((ZR, d), jnp.float32),
            pltpu.VMEM((EPW,), jnp.int32),
            pltpu.VMEM((K, CH), jnp.int32),
            [pltpu.SemaphoreType.DMA] * K,
            [pltpu.SemaphoreType.DMA] * K,
            [pltpu.SemaphoreType.DMA] * K,
        ],
    )
    def k(y_hbm, src_hbm, dst_hbm, out_hbm, acc_sh, rows_v, zb_v, isrc_v, idst_v,
          gsems, ssems, isems):
        cid = lax.axis_index("c")
        sid = lax.axis_index("s")
        wid = sid * NC + cid

        pltpu.sync_copy(src_hbm.at[wid], isrc_v)
        _zero_buf(zb_v, ZR, d)

        def zcp(kk, carry):
            pltpu.sync_copy(zb_v, acc_sh.at[pl.ds(sid * RPT + kk * ZR, ZR)])
            return carry

        lax.fori_loop(0, NZ, zcp, 0)
        plsc.subcore_barrier()

        def _gather_args(ci, b):
            return (y_hbm.at[isrc_v.at[pl.ds(ci * CH, CH)]], rows_v.at[b],
                    gsems[b])

        def _didx_args(ci, b):
            return (dst_hbm.at[wid, ci], idst_v.at[b], isems[b])

        def _scatter_args(b):
            return (rows_v.at[b], acc_sh.at[idst_v.at[b]], ssems[b])

        for b in range(K):
            pltpu.async_copy(*_gather_args(b, b))
            pltpu.async_copy(*_didx_args(b, b))

        def body(g, carry):
            c0 = g * K
            for b in range(K):
                pltpu.make_async_copy(*_gather_args(c0 + b, b)).wait()
                pltpu.make_async_copy(*_didx_args(c0 + b, b)).wait()
                pltpu.async_copy(*_scatter_args(b), add=True)
            for b in range(K):
                cn = c0 + K + b
                pltpu.make_async_copy(*_scatter_args(b)).wait()

                @pl.when(cn < NCHUNK)
                def _():
                    pltpu.async_copy(*_gather_args(cn, b))
                    pltpu.async_copy(*_didx_args(cn, b))

            return carry

        lax.fori_loop(0, NCHUNK // K, body, 0)
        plsc.subcore_barrier()

        pltpu.sync_copy(acc_sh.at[pl.ds(sid * RPT, RPT)], out_hbm.at[cid, sid])

    return k


_sc_pass128 = _make_sc_pass(D1)
_sc_pass16 = _make_sc_pass(D2)

_BM = 1000


def _ns_from(d_ref, which):
    cnt = d_ref[0, which][:, :1] + d_ref[1, which][:, :1]
    return lax.rsqrt(jnp.maximum(cnt, 1.0))


def _tc1(x, W1, degp):
    def body(x_ref, w_ref, d_ref, o_ref):
        ns = _ns_from(d_ref, 0)
        o_ref[...] = jnp.dot(x_ref[...] * ns, w_ref[...],
                             preferred_element_type=jnp.float32)

    return pl.pallas_call(
        body,
        grid=(N // _BM,),
        in_specs=[
            pl.BlockSpec((_BM, D1), lambda m: (m, 0)),
            pl.BlockSpec((D1, D1), lambda m: (0, 0)),
            pl.BlockSpec((2, 2, _BM, 16), lambda m: (0, 0, m, 0)),
        ],
        out_specs=pl.BlockSpec((_BM, D1), lambda m: (m, 0)),
        out_shape=jax.ShapeDtypeStruct((N, D1), jnp.float32),
    )(x, W1, degp)


def _tc2(aggp, degp, b1r, W2):
    def body(a_ref, d_ref, b_ref, w_ref, o_ref):
        agg = a_ref[0] + a_ref[1]
        nd = _ns_from(d_ref, 1)
        ns = _ns_from(d_ref, 0)
        h = jnp.maximum(agg * nd + b_ref[...], 0.0)
        o_ref[...] = jnp.dot(h * ns, w_ref[...],
                             preferred_element_type=jnp.float32)

    return pl.pallas_call(
        body,
        grid=(N // _BM,),
        in_specs=[
            pl.BlockSpec((2, _BM, D1), lambda m: (0, m, 0)),
            pl.BlockSpec((2, 2, _BM, 16), lambda m: (0, 0, m, 0)),
            pl.BlockSpec((1, D1), lambda m: (0, 0)),
            pl.BlockSpec((D1, D2), lambda m: (0, 0)),
        ],
        out_specs=pl.BlockSpec((_BM, D2), lambda m: (m, 0)),
        out_shape=jax.ShapeDtypeStruct((N, D2), jnp.float32),
    )(aggp, degp, b1r, W2)


def _tc3(aggp, degp, b2r):
    def body(a_ref, d_ref, b_ref, o_ref):
        agg = a_ref[0] + a_ref[1]
        nd = _ns_from(d_ref, 1)
        o_ref[...] = agg * nd + b_ref[...]

    return pl.pallas_call(
        body,
        grid=(N // _BM,),
        in_specs=[
            pl.BlockSpec((2, _BM, D2), lambda m: (0, m, 0)),
            pl.BlockSpec((2, 2, _BM, 16), lambda m: (0, 0, m, 0)),
            pl.BlockSpec((1, D2), lambda m: (0, 0)),
        ],
        out_specs=pl.BlockSpec((_BM, D2), lambda m: (m, 0)),
        out_shape=jax.ShapeDtypeStruct((N, D2), jnp.float32),
    )(aggp, degp, b2r)


def kernel(features, edge_index, W1, b1, W2, b2):
    src2 = edge_index[0].reshape(NW, EPW)
    src3 = edge_index[0].reshape(NW, NCHUNK, CH)
    dst3 = edge_index[1].reshape(NW, NCHUNK, CH)
    deg_parts = _sc_degrees(src3, dst3).reshape(NC, 2, N, 16)
    y1s = _tc1(features, W1, deg_parts)
    agg1 = _sc_pass128(y1s, src2, dst3).reshape(NC, N, D1)
    y2s = _tc2(agg1, deg_parts, b1.reshape(1, D1), W2)
    agg2 = _sc_pass16(y2s, src2, dst3).reshape(NC, N, D2)
    return _tc3(agg2, deg_parts, b2.reshape(1, D2))


# speedup vs baseline: 12.4162x; 1.0586x over previous
import functools

import jax
import jax.numpy as jnp
from jax import lax
from jax.experimental import pallas as pl
from jax.experimental.pallas import tpu as pltpu
from jax.experimental.pallas import tpu_sc as plsc

N = 10000
E = 320000
D1 = 128
D2 = 16

NC, NS = 2, 16
NW = NC * NS
CH = 80
EPW = E // NW
NCHUNK = EPW // CH
RPT = N // NS
ZR = 25
NZ = RPT // ZR

_MESH = plsc.VectorSubcoreMesh(core_axis_name="c", subcore_axis_name="s")
_SC_PARAMS = pltpu.CompilerParams(use_tc_tiling_on_sc=False)


def _zero_buf(buf, rows, d):
    z16 = jnp.zeros((16,), jnp.float32)

    def zrow(r, carry):
        for c8 in range(d // 16):
            buf[r, pl.ds(c8 * 16, 16)] = z16
        return carry

    lax.fori_loop(0, rows, zrow, 0)


@functools.partial(
    pl.kernel,
    out_type=jax.ShapeDtypeStruct((NC, 2, NS, RPT, 16), jnp.float32),
    mesh=_MESH,
    compiler_params=_SC_PARAMS,
    scratch_types=[
        pltpu.VMEM_SHARED((N, 16), jnp.float32),
        pltpu.VMEM_SHARED((N, 16), jnp.float32),
        pltpu.VMEM((CH, 16), jnp.float32),
        pltpu.VMEM((ZR, 16), jnp.float32),
        pltpu.VMEM((NCHUNK, CH), jnp.int32),
        pltpu.VMEM((NCHUNK, CH), jnp.int32),
        pltpu.SemaphoreType.DMA,
    ],
)
def _sc_degrees(src_hbm, dst_hbm, out_hbm, do_sh, di_sh, ones_v, zb_v, isrc_v, idst_v, sem):
    cid = lax.axis_index("c")
    sid = lax.axis_index("s")
    wid = sid * NC + cid

    one16 = jnp.ones((16,), jnp.float32)

    def orow(r, carry):
        ones_v[r, pl.ds(0, 16)] = one16
        return carry

    lax.fori_loop(0, CH, orow, 0)
    _zero_buf(zb_v, ZR, 16)

    pltpu.sync_copy(src_hbm.at[wid], isrc_v)
    pltpu.sync_copy(dst_hbm.at[wid], idst_v)

    def zcp(k, carry):
        r0 = sid * RPT + k * ZR
        pltpu.sync_copy(zb_v, do_sh.at[pl.ds(r0, ZR)])
        pltpu.sync_copy(zb_v, di_sh.at[pl.ds(r0, ZR)])
        return carry

    lax.fori_loop(0, NZ, zcp, 0)
    plsc.subcore_barrier()

    def fire(ci, carry):
        pltpu.async_copy(ones_v, do_sh.at[isrc_v.at[ci]], sem, add=True)
        pltpu.async_copy(ones_v, di_sh.at[idst_v.at[ci]], sem, add=True)
        return carry

    lax.fori_loop(0, NCHUNK, fire, 0)

    def drain(ci, carry):
        pltpu.make_async_copy(ones_v, do_sh.at[isrc_v.at[0]], sem).wait()
        return carry

    lax.fori_loop(0, 2 * NCHUNK, drain, 0)
    plsc.subcore_barrier()

    pltpu.sync_copy(do_sh.at[pl.ds(sid * RPT, RPT)], out_hbm.at[cid, 0, sid])
    pltpu.sync_copy(di_sh.at[pl.ds(sid * RPT, RPT)], out_hbm.at[cid, 1, sid])


def _make_sc_pass(d, K):
    @functools.partial(
        pl.kernel,
        out_type=jax.ShapeDtypeStruct((NC, NS, RPT, d), jnp.float32),
        mesh=_MESH,
        compiler_params=_SC_PARAMS,
        scratch_types=[
            pltpu.VMEM_SHARED((N, d), jnp.float32),
            pltpu.VMEM((K, CH, d), jnp.float32),
            pltpu.VMEM((ZR, d), jnp.float32),
            pltpu.VMEM((EPW,), jnp.int32),
            pltpu.VMEM((K, CH), jnp.int32),
            [pltpu.SemaphoreType.DMA] * K,
            [pltpu.SemaphoreType.DMA] * K,
            [pltpu.SemaphoreType.DMA] * K,
        ],
    )
    def k(y_hbm, src_hbm, dst_hbm, out_hbm, acc_sh, rows_v, zb_v, isrc_v, idst_v,
          gsems, ssems, isems):
        cid = lax.axis_index("c")
        sid = lax.axis_index("s")
        wid = sid * NC + cid

        pltpu.sync_copy(src_hbm.at[wid], isrc_v)
        _zero_buf(zb_v, ZR, d)

        def zcp(kk, carry):
            pltpu.sync_copy(zb_v, acc_sh.at[pl.ds(sid * RPT + kk * ZR, ZR)])
            return carry

        lax.fori_loop(0, NZ, zcp, 0)
        plsc.subcore_barrier()

        def _gather_args(ci, b):
            return (y_hbm.at[isrc_v.at[pl.ds(ci * CH, CH)]], rows_v.at[b],
                    gsems[b])

        def _didx_args(ci, b):
            return (dst_hbm.at[wid, ci], idst_v.at[b], isems[b])

        def _scatter_args(b):
            return (rows_v.at[b], acc_sh.at[idst_v.at[b]], ssems[b])

        for b in range(K):
            pltpu.async_copy(*_gather_args(b, b))
            pltpu.async_copy(*_didx_args(b, b))

        def body(g, carry):
            c0 = g * K
            for b in range(K):
                c = c0 + b

                @pl.when(c < NCHUNK)
                def _():
                    pltpu.make_async_copy(*_gather_args(c, b)).wait()
                    pltpu.make_async_copy(*_didx_args(c, b)).wait()
                    pltpu.async_copy(*_scatter_args(b), add=True)

            for b in range(K):
                c = c0 + b
                cn = c + K

                @pl.when(c < NCHUNK)
                def _():
                    pltpu.make_async_copy(*_scatter_args(b)).wait()

                @pl.when(cn < NCHUNK)
                def _():
                    pltpu.async_copy(*_gather_args(cn, b))
                    pltpu.async_copy(*_didx_args(cn, b))

            return carry

        lax.fori_loop(0, -(-NCHUNK // K), body, 0)
        plsc.subcore_barrier()

        pltpu.sync_copy(acc_sh.at[pl.ds(sid * RPT, RPT)], out_hbm.at[cid, sid])

    return k


_sc_pass128 = _make_sc_pass(D1, 3)
_sc_pass16 = _make_sc_pass(D2, 5)

_BM = 1000


def _ns_from(d_ref, which):
    cnt = d_ref[0, which][:, :1] + d_ref[1, which][:, :1]
    return lax.rsqrt(jnp.maximum(cnt, 1.0))


def _tc1(x, W1, degp):
    def body(x_ref, w_ref, d_ref, o_ref):
        ns = _ns_from(d_ref, 0)
        o_ref[...] = jnp.dot(x_ref[...] * ns, w_ref[...],
                             preferred_element_type=jnp.float32)

    return pl.pallas_call(
        body,
        grid=(N // _BM,),
        in_specs=[
            pl.BlockSpec((_BM, D1), lambda m: (m, 0)),
            pl.BlockSpec((D1, D1), lambda m: (0, 0)),
            pl.BlockSpec((2, 2, _BM, 16), lambda m: (0, 0, m, 0)),
        ],
        out_specs=pl.BlockSpec((_BM, D1), lambda m: (m, 0)),
        out_shape=jax.ShapeDtypeStruct((N, D1), jnp.float32),
    )(x, W1, degp)


def _tc2(aggp, degp, b1r, W2):
    def body(a_ref, d_ref, b_ref, w_ref, o_ref):
        agg = a_ref[0] + a_ref[1]
        nd = _ns_from(d_ref, 1)
        ns = _ns_from(d_ref, 0)
        h = jnp.maximum(agg * nd + b_ref[...], 0.0)
        o_ref[...] = jnp.dot(h * ns, w_ref[...],
                             preferred_element_type=jnp.float32)

    return pl.pallas_call(
        body,
        grid=(N // _BM,),
        in_specs=[
            pl.BlockSpec((2, _BM, D1), lambda m: (0, m, 0)),
            pl.BlockSpec((2, 2, _BM, 16), lambda m: (0, 0, m, 0)),
            pl.BlockSpec((1, D1), lambda m: (0, 0)),
            pl.BlockSpec((D1, D2), lambda m: (0, 0)),
        ],
        out_specs=pl.BlockSpec((_BM, D2), lambda m: (m, 0)),
        out_shape=jax.ShapeDtypeStruct((N, D2), jnp.float32),
    )(aggp, degp, b1r, W2)


def _tc3(aggp, degp, b2r):
    def body(a_ref, d_ref, b_ref, o_ref):
        agg = a_ref[0] + a_ref[1]
        nd = _ns_from(d_ref, 1)
        o_ref[...] = agg * nd + b_ref[...]

    return pl.pallas_call(
        body,
        grid=(N // _BM,),
        in_specs=[
            pl.BlockSpec((2, _BM, D2), lambda m: (0, m, 0)),
            pl.BlockSpec((2, 2, _BM, 16), lambda m: (0, 0, m, 0)),
            pl.BlockSpec((1, D2), lambda m: (0, 0)),
        ],
        out_specs=pl.BlockSpec((_BM, D2), lambda m: (m, 0)),
        out_shape=jax.ShapeDtypeStruct((N, D2), jnp.float32),
    )(aggp, degp, b2r)


def kernel(features, edge_index, W1, b1, W2, b2):
    src2 = edge_index[0].reshape(NW, EPW)
    src3 = edge_index[0].reshape(NW, NCHUNK, CH)
    dst3 = edge_index[1].reshape(NW, NCHUNK, CH)
    deg_parts = _sc_degrees(src3, dst3).reshape(NC, 2, N, 16)
    y1s = _tc1(features, W1, deg_parts)
    agg1 = _sc_pass128(y1s, src2, dst3).reshape(NC, N, D1)
    y2s = _tc2(agg1, deg_parts, b1.reshape(1, D1), W2)
    agg2 = _sc_pass16(y2s, src2, dst3).reshape(NC, N, D2)
    return _tc3(agg2, deg_parts, b2.reshape(1, D2))


# speedup vs baseline: 12.7288x; 1.0252x over previous
import functools

import jax
import jax.numpy as jnp
from jax import lax
from jax.experimental import pallas as pl
from jax.experimental.pallas import tpu as pltpu
from jax.experimental.pallas import tpu_sc as plsc

N = 10000
E = 320000
D1 = 128
D2 = 16

NC, NS = 2, 16
NW = NC * NS
CH = 80
EPW = E // NW
NCHUNK = EPW // CH
RPT = N // NS
ZR = 25
NZ = RPT // ZR

_MESH = plsc.VectorSubcoreMesh(core_axis_name="c", subcore_axis_name="s")
_SC_PARAMS = pltpu.CompilerParams(use_tc_tiling_on_sc=False,
                                  skip_device_barrier=True)


def _zero_buf(buf, rows, d):
    z16 = jnp.zeros((16,), jnp.float32)

    def zrow(r, carry):
        for c8 in range(d // 16):
            buf[r, pl.ds(c8 * 16, 16)] = z16
        return carry

    lax.fori_loop(0, rows, zrow, 0)


@functools.partial(
    pl.kernel,
    out_type=jax.ShapeDtypeStruct((NC, 2, NS, RPT, 16), jnp.float32),
    mesh=_MESH,
    compiler_params=_SC_PARAMS,
    scratch_types=[
        pltpu.VMEM_SHARED((N, 16), jnp.float32),
        pltpu.VMEM_SHARED((N, 16), jnp.float32),
        pltpu.VMEM((CH, 16), jnp.float32),
        pltpu.VMEM((ZR, 16), jnp.float32),
        pltpu.VMEM((NCHUNK, CH), jnp.int32),
        pltpu.VMEM((NCHUNK, CH), jnp.int32),
        pltpu.SemaphoreType.DMA,
    ],
)
def _sc_degrees(src_hbm, dst_hbm, out_hbm, do_sh, di_sh, ones_v, zb_v, isrc_v, idst_v, sem):
    cid = lax.axis_index("c")
    sid = lax.axis_index("s")
    wid = sid * NC + cid

    one16 = jnp.ones((16,), jnp.float32)

    def orow(r, carry):
        ones_v[r, pl.ds(0, 16)] = one16
        return carry

    lax.fori_loop(0, CH, orow, 0)
    _zero_buf(zb_v, ZR, 16)

    pltpu.sync_copy(src_hbm.at[wid], isrc_v)
    pltpu.sync_copy(dst_hbm.at[wid], idst_v)

    def zcp(k, carry):
        r0 = sid * RPT + k * ZR
        pltpu.sync_copy(zb_v, do_sh.at[pl.ds(r0, ZR)])
        pltpu.sync_copy(zb_v, di_sh.at[pl.ds(r0, ZR)])
        return carry

    lax.fori_loop(0, NZ, zcp, 0)
    plsc.subcore_barrier()

    def fire(ci, carry):
        pltpu.async_copy(ones_v, do_sh.at[isrc_v.at[ci]], sem, add=True)
        pltpu.async_copy(ones_v, di_sh.at[idst_v.at[ci]], sem, add=True)
        return carry

    lax.fori_loop(0, NCHUNK, fire, 0)

    def drain(ci, carry):
        pltpu.make_async_copy(ones_v, do_sh.at[isrc_v.at[0]], sem).wait()
        return carry

    lax.fori_loop(0, 2 * NCHUNK, drain, 0)
    plsc.subcore_barrier()

    pltpu.sync_copy(do_sh.at[pl.ds(sid * RPT, RPT)], out_hbm.at[cid, 0, sid])
    pltpu.sync_copy(di_sh.at[pl.ds(sid * RPT, RPT)], out_hbm.at[cid, 1, sid])


def _make_sc_pass(d, ch, K):
    nchunk = EPW // ch

    @functools.partial(
        pl.kernel,
        out_type=jax.ShapeDtypeStruct((NC, NS, RPT, d), jnp.float32),
        mesh=_MESH,
        compiler_params=_SC_PARAMS,
        scratch_types=[
            pltpu.VMEM_SHARED((N, d), jnp.float32),
            pltpu.VMEM((K, ch, d), jnp.float32),
            pltpu.VMEM((ZR, d), jnp.float32),
            pltpu.VMEM((EPW,), jnp.int32),
            pltpu.VMEM((K, ch), jnp.int32),
            [pltpu.SemaphoreType.DMA] * K,
            [pltpu.SemaphoreType.DMA] * K,
            [pltpu.SemaphoreType.DMA] * K,
        ],
    )
    def k(y_hbm, src_hbm, dst_hbm, out_hbm, acc_sh, rows_v, zb_v, isrc_v, idst_v,
          gsems, ssems, isems):
        cid = lax.axis_index("c")
        sid = lax.axis_index("s")
        wid = sid * NC + cid

        pltpu.sync_copy(src_hbm.at[wid], isrc_v)
        _zero_buf(zb_v, ZR, d)

        def zcp(kk, carry):
            pltpu.sync_copy(zb_v, acc_sh.at[pl.ds(sid * RPT + kk * ZR, ZR)])
            return carry

        lax.fori_loop(0, NZ, zcp, 0)
        plsc.subcore_barrier()

        def _gather_args(ci, b):
            return (y_hbm.at[isrc_v.at[pl.ds(ci * ch, ch)]], rows_v.at[b],
                    gsems[b])

        def _didx_args(ci, b):
            return (dst_hbm.at[wid, ci], idst_v.at[b], isems[b])

        def _scatter_args(b):
            return (rows_v.at[b], acc_sh.at[idst_v.at[b]], ssems[b])

        for b in range(K):
            pltpu.async_copy(*_gather_args(b, b))
            pltpu.async_copy(*_didx_args(b, b))

        def body(g, carry):
            c0 = g * K
            for b in range(K):
                c = c0 + b

                @pl.when(c < nchunk)
                def _():
                    pltpu.make_async_copy(*_gather_args(c, b)).wait()
                    pltpu.make_async_copy(*_didx_args(c, b)).wait()
                    pltpu.async_copy(*_scatter_args(b), add=True)

            for b in range(K):
                c = c0 + b
                cn = c + K

                @pl.when(c < nchunk)
                def _():
                    pltpu.make_async_copy(*_scatter_args(b)).wait()

                @pl.when(cn < nchunk)
                def _():
                    pltpu.async_copy(*_gather_args(cn, b))
                    pltpu.async_copy(*_didx_args(cn, b))

            return carry

        lax.fori_loop(0, -(-nchunk // K), body, 0)
        plsc.subcore_barrier()

        pltpu.sync_copy(acc_sh.at[pl.ds(sid * RPT, RPT)], out_hbm.at[cid, sid])

    return k


_sc_pass128 = _make_sc_pass(D1, 40, 5)
_sc_pass16 = _make_sc_pass(D2, 80, 5)

_BM = 1000


def _ns_from(d_ref, which):
    cnt = d_ref[0, which][:, :1] + d_ref[1, which][:, :1]
    return lax.rsqrt(jnp.maximum(cnt, 1.0))


def _tc1(x, W1, degp):
    def body(x_ref, w_ref, d_ref, o_ref):
        ns = _ns_from(d_ref, 0)
        o_ref[...] = jnp.dot(x_ref[...] * ns, w_ref[...],
                             preferred_element_type=jnp.float32)

    return pl.pallas_call(
        body,
        grid=(N // _BM,),
        in_specs=[
            pl.BlockSpec((_BM, D1), lambda m: (m, 0)),
            pl.BlockSpec((D1, D1), lambda m: (0, 0)),
            pl.BlockSpec((2, 2, _BM, 16), lambda m: (0, 0, m, 0)),
        ],
        out_specs=pl.BlockSpec((_BM, D1), lambda m: (m, 0)),
        out_shape=jax.ShapeDtypeStruct((N, D1), jnp.float32),
    )(x, W1, degp)


def _tc2(aggp, degp, b1r, W2):
    def body(a_ref, d_ref, b_ref, w_ref, o_ref):
        agg = a_ref[0] + a_ref[1]
        nd = _ns_from(d_ref, 1)
        ns = _ns_from(d_ref, 0)
        h = jnp.maximum(agg * nd + b_ref[...], 0.0)
        o_ref[...] = jnp.dot(h * ns, w_ref[...],
                             preferred_element_type=jnp.float32)

    return pl.pallas_call(
        body,
        grid=(N // _BM,),
        in_specs=[
            pl.BlockSpec((2, _BM, D1), lambda m: (0, m, 0)),
            pl.BlockSpec((2, 2, _BM, 16), lambda m: (0, 0, m, 0)),
            pl.BlockSpec((1, D1), lambda m: (0, 0)),
            pl.BlockSpec((D1, D2), lambda m: (0, 0)),
        ],
        out_specs=pl.BlockSpec((_BM, D2), lambda m: (m, 0)),
        out_shape=jax.ShapeDtypeStruct((N, D2), jnp.float32),
    )(aggp, degp, b1r, W2)


def _tc3(aggp, degp, b2r):
    def body(a_ref, d_ref, b_ref, o_ref):
        agg = a_ref[0] + a_ref[1]
        nd = _ns_from(d_ref, 1)
        o_ref[...] = agg * nd + b_ref[...]

    return pl.pallas_call(
        body,
        grid=(N // _BM,),
        in_specs=[
            pl.BlockSpec((2, _BM, D2), lambda m: (0, m, 0)),
            pl.BlockSpec((2, 2, _BM, 16), lambda m: (0, 0, m, 0)),
            pl.BlockSpec((1, D2), lambda m: (0, 0)),
        ],
        out_specs=pl.BlockSpec((_BM, D2), lambda m: (m, 0)),
        out_shape=jax.ShapeDtypeStruct((N, D2), jnp.float32),
    )(aggp, degp, b2r)


def kernel(features, edge_index, W1, b1, W2, b2):
    src2 = edge_index[0].reshape(NW, EPW)
    src3 = edge_index[0].reshape(NW, NCHUNK, CH)
    dst3 = edge_index[1].reshape(NW, NCHUNK, CH)
    dst40 = edge_index[1].reshape(NW, EPW // 40, 40)
    deg_parts = _sc_degrees(src3, dst3).reshape(NC, 2, N, 16)
    y1s = _tc1(features, W1, deg_parts)
    agg1 = _sc_pass128(y1s, src2, dst40).reshape(NC, N, D1)
    y2s = _tc2(agg1, deg_parts, b1.reshape(1, D1), W2)
    agg2 = _sc_pass16(y2s, src2, dst3).reshape(NC, N, D2)
    return _tc3(agg2, deg_parts, b2.reshape(1, D2))


# speedup vs baseline: 12.8686x; 1.0110x over previous
import functools

import jax
import jax.numpy as jnp
from jax import lax
from jax.experimental import pallas as pl
from jax.experimental.pallas import tpu as pltpu
from jax.experimental.pallas import tpu_sc as plsc

N = 10000
E = 320000
D1 = 128
D2 = 16

NC, NS = 2, 16
NW = NC * NS
CH = 80
EPW = E // NW
NCHUNK = EPW // CH
RPT = N // NS
ZR = 25
NZ = RPT // ZR

_MESH = plsc.VectorSubcoreMesh(core_axis_name="c", subcore_axis_name="s")
_SC_PARAMS = pltpu.CompilerParams(use_tc_tiling_on_sc=False,
                                  skip_device_barrier=True)


def _zero_buf(buf, rows, d):
    z16 = jnp.zeros((16,), jnp.float32)

    def zrow(r, carry):
        for c8 in range(d // 16):
            buf[r, pl.ds(c8 * 16, 16)] = z16
        return carry

    lax.fori_loop(0, rows, zrow, 0)


@functools.partial(
    pl.kernel,
    out_type=jax.ShapeDtypeStruct((NC, 2, NS, RPT, 16), jnp.float32),
    mesh=_MESH,
    compiler_params=_SC_PARAMS,
    scratch_types=[
        pltpu.VMEM_SHARED((N, 16), jnp.float32),
        pltpu.VMEM_SHARED((N, 16), jnp.float32),
        pltpu.VMEM((CH, 16), jnp.float32),
        pltpu.VMEM((ZR, 16), jnp.float32),
        pltpu.VMEM((NCHUNK, CH), jnp.int32),
        pltpu.VMEM((NCHUNK, CH), jnp.int32),
        pltpu.SemaphoreType.DMA,
    ],
)
def _sc_degrees(src_hbm, dst_hbm, out_hbm, do_sh, di_sh, ones_v, zb_v, isrc_v, idst_v, sem):
    cid = lax.axis_index("c")
    sid = lax.axis_index("s")
    wid = sid * NC + cid

    one16 = jnp.ones((16,), jnp.float32)

    def orow(r, carry):
        ones_v[r, pl.ds(0, 16)] = one16
        return carry

    lax.fori_loop(0, CH, orow, 0)
    _zero_buf(zb_v, ZR, 16)

    pltpu.sync_copy(src_hbm.at[wid], isrc_v)
    pltpu.sync_copy(dst_hbm.at[wid], idst_v)

    def zcp(k, carry):
        r0 = sid * RPT + k * ZR
        pltpu.sync_copy(zb_v, do_sh.at[pl.ds(r0, ZR)])
        pltpu.sync_copy(zb_v, di_sh.at[pl.ds(r0, ZR)])
        return carry

    lax.fori_loop(0, NZ, zcp, 0)
    plsc.subcore_barrier()

    def fire(ci, carry):
        pltpu.async_copy(ones_v, do_sh.at[isrc_v.at[ci]], sem, add=True)
        pltpu.async_copy(ones_v, di_sh.at[idst_v.at[ci]], sem, add=True)
        return carry

    lax.fori_loop(0, NCHUNK, fire, 0)

    def drain(ci, carry):
        pltpu.make_async_copy(ones_v, do_sh.at[isrc_v.at[0]], sem).wait()
        return carry

    lax.fori_loop(0, 2 * NCHUNK, drain, 0)
    plsc.subcore_barrier()

    pltpu.sync_copy(do_sh.at[pl.ds(sid * RPT, RPT)], out_hbm.at[cid, 0, sid])
    pltpu.sync_copy(di_sh.at[pl.ds(sid * RPT, RPT)], out_hbm.at[cid, 1, sid])


def _make_sc_pass(d, ch, K):
    nchunk = EPW // ch

    @functools.partial(
        pl.kernel,
        out_type=jax.ShapeDtypeStruct((NC, NS, RPT, d), jnp.float32),
        mesh=_MESH,
        compiler_params=_SC_PARAMS,
        scratch_types=[
            pltpu.VMEM_SHARED((N, d), jnp.float32),
            pltpu.VMEM((K, ch, d), jnp.float32),
            pltpu.VMEM((ZR, d), jnp.float32),
            pltpu.VMEM((EPW,), jnp.int32),
            pltpu.VMEM((K, ch), jnp.int32),
            [pltpu.SemaphoreType.DMA] * K,
            [pltpu.SemaphoreType.DMA] * K,
            [pltpu.SemaphoreType.DMA] * K,
        ],
    )
    def k(y_hbm, src_hbm, dst_hbm, out_hbm, acc_sh, rows_v, zb_v, isrc_v, idst_v,
          gsems, ssems, isems):
        cid = lax.axis_index("c")
        sid = lax.axis_index("s")
        wid = sid * NC + cid

        pltpu.sync_copy(src_hbm.at[wid], isrc_v)
        _zero_buf(zb_v, ZR, d)

        def zcp(kk, carry):
            pltpu.sync_copy(zb_v, acc_sh.at[pl.ds(sid * RPT + kk * ZR, ZR)])
            return carry

        lax.fori_loop(0, NZ, zcp, 0)
        plsc.subcore_barrier()

        def _gather_args(ci, b):
            return (y_hbm.at[isrc_v.at[pl.ds(ci * ch, ch)]], rows_v.at[b],
                    gsems[b])

        def _didx_args(ci, b):
            return (dst_hbm.at[wid, ci], idst_v.at[b], isems[b])

        def _scatter_args(b):
            return (rows_v.at[b], acc_sh.at[idst_v.at[b]], ssems[b])

        for b in range(K):
            pltpu.async_copy(*_gather_args(b, b))
            pltpu.async_copy(*_didx_args(b, b))

        def body(g, carry):
            c0 = g * K
            for b in range(K):
                c = c0 + b

                @pl.when(c < nchunk)
                def _():
                    pltpu.make_async_copy(*_gather_args(c, b)).wait()
                    pltpu.make_async_copy(*_didx_args(c, b)).wait()
                    pltpu.async_copy(*_scatter_args(b), add=True)

            for b in range(K):
                c = c0 + b
                cn = c + K

                @pl.when(c < nchunk)
                def _():
                    pltpu.make_async_copy(*_scatter_args(b)).wait()

                @pl.when(cn < nchunk)
                def _():
                    pltpu.async_copy(*_gather_args(cn, b))
                    pltpu.async_copy(*_didx_args(cn, b))

            return carry

        lax.fori_loop(0, -(-nchunk // K), body, 0)
        plsc.subcore_barrier()

        pltpu.sync_copy(acc_sh.at[pl.ds(sid * RPT, RPT)], out_hbm.at[cid, sid])

    return k


_sc_pass128 = _make_sc_pass(D1, 40, 7)
_sc_pass16 = _make_sc_pass(D2, 80, 5)

_BM = 1000


def _ns_from(d_ref, which):
    cnt = d_ref[0, which][:, :1] + d_ref[1, which][:, :1]
    return lax.rsqrt(jnp.maximum(cnt, 1.0))


def _tc1(x, W1, degp):
    def body(x_ref, w_ref, d_ref, o_ref):
        ns = _ns_from(d_ref, 0)
        o_ref[...] = jnp.dot(x_ref[...] * ns, w_ref[...],
                             preferred_element_type=jnp.float32)

    return pl.pallas_call(
        body,
        grid=(N // _BM,),
        in_specs=[
            pl.BlockSpec((_BM, D1), lambda m: (m, 0)),
            pl.BlockSpec((D1, D1), lambda m: (0, 0)),
            pl.BlockSpec((2, 2, _BM, 16), lambda m: (0, 0, m, 0)),
        ],
        out_specs=pl.BlockSpec((_BM, D1), lambda m: (m, 0)),
        out_shape=jax.ShapeDtypeStruct((N, D1), jnp.float32),
    )(x, W1, degp)


def _tc2(aggp, degp, b1r, W2):
    def body(a_ref, d_ref, b_ref, w_ref, o_ref):
        agg = a_ref[0] + a_ref[1]
        nd = _ns_from(d_ref, 1)
        ns = _ns_from(d_ref, 0)
        h = jnp.maximum(agg * nd + b_ref[...], 0.0)
        o_ref[...] = jnp.dot(h * ns, w_ref[...],
                             preferred_element_type=jnp.float32)

    return pl.pallas_call(
        body,
        grid=(N // _BM,),
        in_specs=[
            pl.BlockSpec((2, _BM, D1), lambda m: (0, m, 0)),
            pl.BlockSpec((2, 2, _BM, 16), lambda m: (0, 0, m, 0)),
            pl.BlockSpec((1, D1), lambda m: (0, 0)),
            pl.BlockSpec((D1, D2), lambda m: (0, 0)),
        ],
        out_specs=pl.BlockSpec((_BM, D2), lambda m: (m, 0)),
        out_shape=jax.ShapeDtypeStruct((N, D2), jnp.float32),
    )(aggp, degp, b1r, W2)


def _tc3(aggp, degp, b2r):
    def body(a_ref, d_ref, b_ref, o_ref):
        agg = a_ref[0] + a_ref[1]
        nd = _ns_from(d_ref, 1)
        o_ref[...] = agg * nd + b_ref[...]

    return pl.pallas_call(
        body,
        grid=(N // _BM,),
        in_specs=[
            pl.BlockSpec((2, _BM, D2), lambda m: (0, m, 0)),
            pl.BlockSpec((2, 2, _BM, 16), lambda m: (0, 0, m, 0)),
            pl.BlockSpec((1, D2), lambda m: (0, 0)),
        ],
        out_specs=pl.BlockSpec((_BM, D2), lambda m: (m, 0)),
        out_shape=jax.ShapeDtypeStruct((N, D2), jnp.float32),
    )(aggp, degp, b2r)


def kernel(features, edge_index, W1, b1, W2, b2):
    src2 = edge_index[0].reshape(NW, EPW)
    src3 = edge_index[0].reshape(NW, NCHUNK, CH)
    dst3 = edge_index[1].reshape(NW, NCHUNK, CH)
    dst40 = edge_index[1].reshape(NW, EPW // 40, 40)
    deg_parts = _sc_degrees(src3, dst3).reshape(NC, 2, N, 16)
    y1s = _tc1(features, W1, deg_parts)
    agg1 = _sc_pass128(y1s, src2, dst40).reshape(NC, N, D1)
    y2s = _tc2(agg1, deg_parts, b1.reshape(1, D1), W2)
    agg2 = _sc_pass16(y2s, src2, dst3).reshape(NC, N, D2)
    return _tc3(agg2, deg_parts, b2.reshape(1, D2))
